# baseline (speedup 1.0000x reference)
import jax
import jax.numpy as jnp
from jax.experimental import pallas as pl
from jax.experimental.pallas import tpu as pltpu

_BM = 200
_NBUF = 3


def _body(x_ref, w_ref, b_ref, a_hbm, o_ref, abuf, h_ref, sem):
    n = x_ref.shape[0]
    nsteps = n // _BM
    ngroups = nsteps // _NBUF
    tail = nsteps - ngroups * _NBUF

    def copy_in(i, slot):
        return pltpu.make_async_copy(
            a_hbm.at[pl.ds(i * _BM, _BM), :], abuf.at[slot], sem.at[slot]
        )

    for s in range(_NBUF):
        copy_in(s, s).start()

    h = jax.lax.dot_general(
        x_ref[...].astype(jnp.bfloat16), w_ref[...].astype(jnp.bfloat16),
        dimension_numbers=(((1,), (1,)), ((), ())),
        preferred_element_type=jnp.float32,
    )
    h_ref[...] = (h + b_ref[...]).astype(jnp.bfloat16)

    def do_step(i, s):
        copy_in(i, s).wait()
        a = abuf[s].astype(jnp.bfloat16)
        o_ref[pl.ds(i * _BM, _BM), :] = jnp.dot(
            a, h_ref[...], preferred_element_type=jnp.float32
        )
        nxt = i + _NBUF

        @pl.when(nxt < nsteps)
        def _():
            copy_in(nxt, s).start()

    def group(g, carry):
        base = g * _NBUF
        for s in range(_NBUF):
            do_step(base + s, s)
        return carry

    jax.lax.fori_loop(0, ngroups, group, 0)
    for t in range(tail):
        do_step(ngroups * _NBUF + t, t)


@jax.jit
def kernel(x, time_adj, W, b):
    n, d_in = x.shape
    d_out = W.shape[0]
    b2 = b.reshape(1, d_out)
    return pl.pallas_call(
        _body,
        in_specs=[
            pl.BlockSpec((n, d_in), lambda: (0, 0)),
            pl.BlockSpec((d_out, d_in), lambda: (0, 0)),
            pl.BlockSpec((1, d_out), lambda: (0, 0)),
            pl.BlockSpec(memory_space=pl.ANY),
        ],
        out_specs=pl.BlockSpec((n, d_out), lambda: (0, 0)),
        out_shape=jax.ShapeDtypeStruct((n, d_out), jnp.float32),
        scratch_shapes=[
            pltpu.VMEM((_NBUF, _BM, n), jnp.float32),
            pltpu.VMEM((n, d_out), jnp.bfloat16),
            pltpu.SemaphoreType.DMA((_NBUF,)),
        ],
        compiler_params=pltpu.CompilerParams(
            vmem_limit_bytes=100 * 1024 * 1024,
        ),
    )(x, W, b2, time_adj)
